# baseline (speedup 1.0000x reference)
import jax
import jax.numpy as jnp
from jax import lax
from jax.experimental import pallas as pl
from jax.experimental.pallas import tpu as pltpu
from jax.experimental.pallas import tpu_sc as plsc

N = 10000
DEG = 32
E = N * DEG
C = 128
H = 32

NC = 2
NS = 16
NW = NC * NS
L = 16

EDGES_PER_W = E // NW
CHUNK = 80
NCHUNKS = EDGES_PER_W // CHUNK
NPAIR = (NCHUNKS - 1) // 2


def _sc_gather_body(idx_hbm, x0_hbm, x1_hbm, table_hbm, g_hbm, agg_hbm,
                    idx_v, gb0, gb1, ab0, ab1, x0_v, x1_v, sem0, sem1):
    wid = lax.axis_index("s") * NC + lax.axis_index("c")
    base = wid * EDGES_PER_W

    pltpu.sync_copy(idx_hbm.at[pl.ds(base, EDGES_PER_W)], idx_v)
    pltpu.sync_copy(x0_hbm, x0_v)
    pltpu.sync_copy(x1_hbm, x1_v)

    lane = lax.iota(jnp.int32, L)

    def start(ci, gb, sem):
        pltpu.async_copy(table_hbm.at[idx_v.at[pl.ds(ci * CHUNK, CHUNK)]], gb, sem)

    def finish(ci, gb, ab, sem):
        cb = base + ci * CHUNK
        for g in range(CHUNK // L):
            jv = idx_v[pl.ds(ci * CHUNK + g * L, L)]
            ev = cb + g * L + lane
            dv = lax.shift_right_logical(ev, 5)
            xj0 = plsc.load_gather(x0_v, [jv])
            xj1 = plsc.load_gather(x1_v, [jv])
            xi0 = plsc.load_gather(x0_v, [dv])
            xi1 = plsc.load_gather(x1_v, [dv])
            lv = g * L + lane
            zero = jnp.zeros((L,), jnp.int32)
            plsc.store_scatter(ab, [lv, zero], xj0)
            plsc.store_scatter(ab, [lv, zero + 1], xj1)
            plsc.store_scatter(ab, [lv, zero + 2], xi0)
            plsc.store_scatter(ab, [lv, zero + 3], xi1)
        pltpu.make_async_copy(table_hbm.at[idx_v.at[pl.ds(ci * CHUNK, CHUNK)]],
                              gb, sem).wait()
        pltpu.sync_copy(gb, g_hbm.at[pl.ds(cb, CHUNK)])
        pltpu.sync_copy(ab, agg_hbm.at[pl.ds(cb, CHUNK)])

    start(0, gb0, sem0)

    def pair(oi, carry):
        c0 = 2 * oi
        start(c0 + 1, gb1, sem1)
        finish(c0, gb0, ab0, sem0)
        start(c0 + 2, gb0, sem0)
        finish(c0 + 1, gb1, ab1, sem1)
        return carry

    lax.fori_loop(0, NPAIR, pair, 0)
    finish(NCHUNKS - 1, gb0, ab0, sem0)


@jax.jit
def _sc_gather(neighbors_index, x0, x1, table):
    kern = pl.kernel(
        _sc_gather_body,
        out_type=(
            jax.ShapeDtypeStruct((E, C), jnp.bfloat16),
            jax.ShapeDtypeStruct((E, 4), jnp.float32),
        ),
        mesh=plsc.VectorSubcoreMesh(core_axis_name="c", subcore_axis_name="s"),
        compiler_params=pltpu.CompilerParams(
            use_tc_tiling_on_sc=False, needs_layout_passes=False),
        scratch_types=[
            pltpu.VMEM((EDGES_PER_W,), jnp.int32),
            pltpu.VMEM((CHUNK, C), jnp.bfloat16),
            pltpu.VMEM((CHUNK, C), jnp.bfloat16),
            pltpu.VMEM((CHUNK, 4), jnp.float32),
            pltpu.VMEM((CHUNK, 4), jnp.float32),
            pltpu.VMEM((N,), jnp.float32),
            pltpu.VMEM((N,), jnp.float32),
            pltpu.SemaphoreType.DMA,
            pltpu.SemaphoreType.DMA,
        ],
    )
    return kern(neighbors_index, x0, x1, table)


BN = 400
BE = BN * DEG
NBLK = N // BN


def _tc_body(agg_ref, g_ref, w1_ref, b1_ref, w2_ref, b2_ref, out_ref):
    agg = agg_ref[...]
    h = jnp.dot(agg, w1_ref[...], preferred_element_type=jnp.float32) + b1_ref[...]
    h = 0.5 * h * (1.0 + lax.erf(h * 0.7071067811865475))
    rep = jnp.dot(h, w2_ref[...], preferred_element_type=jnp.float32) + b2_ref[...]
    prod = rep * g_ref[...].astype(jnp.float32)
    out_ref[...] = prod.reshape(BN, DEG, C).sum(axis=1)


@jax.jit
def _tc_mlp(agg, g, w1, b1r, w2, b2r):
    return pl.pallas_call(
        _tc_body,
        grid=(NBLK,),
        in_specs=[
            pl.BlockSpec((BE, 4), lambda i: (i, 0)),
            pl.BlockSpec((BE, C), lambda i: (i, 0)),
            pl.BlockSpec((4, H), lambda i: (0, 0)),
            pl.BlockSpec((1, H), lambda i: (0, 0)),
            pl.BlockSpec((H, C), lambda i: (0, 0)),
            pl.BlockSpec((1, C), lambda i: (0, 0)),
        ],
        out_specs=pl.BlockSpec((BN, C), lambda i: (i, 0)),
        out_shape=jax.ShapeDtypeStruct((N, C), jnp.float32),
    )(agg, g, w1, b1r, w2, b2r)


def kernel(x_in, neighbors_index, neighbors_row_splits, in_features, W1, b1, W2, b2):
    table = in_features[0].astype(jnp.bfloat16)
    x0 = x_in[:, 0]
    x1 = x_in[:, 1]
    scale = jnp.float32(1.0 / DEG)
    g, agg = _sc_gather(neighbors_index, x0, x1, table)
    out = _tc_mlp(agg, g, W1, b1.reshape(1, H),
                  W2 * scale, (b2 * scale).reshape(1, C))
    return out[None]


# speedup vs baseline: 2.8102x; 2.8102x over previous
import jax
import jax.numpy as jnp
from jax import lax
from jax.experimental import pallas as pl
from jax.experimental.pallas import tpu as pltpu
from jax.experimental.pallas import tpu_sc as plsc

N = 10000
DEG = 32
E = N * DEG
C = 128
H = 32

NC = 2
NS = 16
NW = NC * NS
L = 16

EDGES_PER_W = E // NW
CHUNK = 80
NCHUNKS = EDGES_PER_W // CHUNK
NPAIR = (NCHUNKS - 1) // 2


def _sc_gather_body(idx_hbm, x0_hbm, x1_hbm, table_hbm, g_hbm, p_hbm,
                    idx_v, gb0, gb1, db0, db1, ab0, ab1, x0_v, x1_v,
                    sem0, sem1, wsem0, wsem1):
    wid = lax.axis_index("s") * NC + lax.axis_index("c")
    base = wid * EDGES_PER_W

    pltpu.sync_copy(idx_hbm.at[pl.ds(base, EDGES_PER_W)], idx_v)
    pltpu.sync_copy(x0_hbm, x0_v)
    pltpu.sync_copy(x1_hbm, x1_v)

    lane = lax.iota(jnp.int32, L)

    def start(ci, gb, sem):
        pltpu.async_copy(table_hbm.at[idx_v.at[pl.ds(ci * CHUNK, CHUNK)]], gb, sem)

    def drain(gb, db, ab, wsem):
        pltpu.make_async_copy(gb, g_hbm.at[db], wsem).wait()
        pltpu.make_async_copy(ab, p_hbm.at[pl.ds(0, CHUNK * 4)], wsem).wait()

    def finish(ci, gb, db, ab, sem, wsem):
        cb = base + ci * CHUNK
        for g in range(CHUNK // L):
            jv = idx_v[pl.ds(ci * CHUNK + g * L, L)]
            ev = cb + g * L + lane
            nv = lax.shift_right_logical(ev, 5)
            kv = lax.bitwise_and(ev, 31)
            db[pl.ds(g * L, L)] = kv * N + nv
            xj0 = plsc.load_gather(x0_v, [jv])
            xj1 = plsc.load_gather(x1_v, [jv])
            xi0 = plsc.load_gather(x0_v, [nv])
            xi1 = plsc.load_gather(x1_v, [nv])
            lv = (g * L + lane) * 4
            plsc.store_scatter(ab, [lv], xj0)
            plsc.store_scatter(ab, [lv + 1], xj1)
            plsc.store_scatter(ab, [lv + 2], xi0)
            plsc.store_scatter(ab, [lv + 3], xi1)
        pltpu.make_async_copy(table_hbm.at[idx_v.at[pl.ds(ci * CHUNK, CHUNK)]],
                              gb, sem).wait()
        pltpu.async_copy(gb, g_hbm.at[db], wsem)
        pltpu.async_copy(ab, p_hbm.at[pl.ds(cb * 4, CHUNK * 4)], wsem)

    start(0, gb0, sem0)
    start(1, gb1, sem1)
    finish(0, gb0, db0, ab0, sem0, wsem0)
    finish(1, gb1, db1, ab1, sem1, wsem1)

    def pair(oi, carry):
        c0 = 2 * oi + 2
        drain(gb0, db0, ab0, wsem0)
        start(c0, gb0, sem0)
        drain(gb1, db1, ab1, wsem1)
        start(c0 + 1, gb1, sem1)
        finish(c0, gb0, db0, ab0, sem0, wsem0)
        finish(c0 + 1, gb1, db1, ab1, sem1, wsem1)
        return carry

    lax.fori_loop(0, NPAIR - 1, pair, 0)
    drain(gb0, db0, ab0, wsem0)
    start(NCHUNKS - 1, gb0, sem0)
    finish(NCHUNKS - 1, gb0, db0, ab0, sem0, wsem0)
    drain(gb0, db0, ab0, wsem0)
    drain(gb1, db1, ab1, wsem1)


@jax.jit
def _sc_gather(neighbors_index, x0, x1, table):
    kern = pl.kernel(
        _sc_gather_body,
        out_type=(
            jax.ShapeDtypeStruct((E, C), jnp.float32),
            jax.ShapeDtypeStruct((4 * E,), jnp.float32),
        ),
        mesh=plsc.VectorSubcoreMesh(core_axis_name="c", subcore_axis_name="s"),
        compiler_params=pltpu.CompilerParams(
            use_tc_tiling_on_sc=False, needs_layout_passes=False),
        scratch_types=[
            pltpu.VMEM((EDGES_PER_W,), jnp.int32),
            pltpu.VMEM((CHUNK, C), jnp.float32),
            pltpu.VMEM((CHUNK, C), jnp.float32),
            pltpu.VMEM((CHUNK,), jnp.int32),
            pltpu.VMEM((CHUNK,), jnp.int32),
            pltpu.VMEM((CHUNK * 4,), jnp.float32),
            pltpu.VMEM((CHUNK * 4,), jnp.float32),
            pltpu.VMEM((N,), jnp.float32),
            pltpu.VMEM((N,), jnp.float32),
            pltpu.SemaphoreType.DMA,
            pltpu.SemaphoreType.DMA,
            pltpu.SemaphoreType.DMA,
            pltpu.SemaphoreType.DMA,
        ],
    )
    return kern(neighbors_index, x0, x1, table)


BN = 400
NBLK = N // BN


def _tc_body(p_ref, g_ref, w1_ref, b1_ref, w2_ref, b2_ref, out_ref):
    hh = jnp.dot(p_ref[...], w1_ref[...], preferred_element_type=jnp.float32)
    hh = hh + b1_ref[...]
    hh = 0.5 * hh * (1.0 + lax.erf(hh * 0.7071067811865475))
    w2 = w2_ref[...]
    b2 = b2_ref[...]
    acc = jnp.zeros((BN, C), jnp.float32)
    for k in range(DEG):
        h_k = hh[:, k * H:(k + 1) * H]
        rep_k = jnp.dot(h_k, w2, preferred_element_type=jnp.float32) + b2
        acc = acc + rep_k * g_ref[k]
    out_ref[...] = acc


@jax.jit
def _tc_mlp(p, g3, w1big, b1big, w2, b2r):
    return pl.pallas_call(
        _tc_body,
        grid=(NBLK,),
        in_specs=[
            pl.BlockSpec((BN, 128), lambda i: (i, 0)),
            pl.BlockSpec((DEG, BN, C), lambda i: (0, i, 0)),
            pl.BlockSpec((128, DEG * H), lambda i: (0, 0)),
            pl.BlockSpec((1, DEG * H), lambda i: (0, 0)),
            pl.BlockSpec((H, C), lambda i: (0, 0)),
            pl.BlockSpec((1, C), lambda i: (0, 0)),
        ],
        out_specs=pl.BlockSpec((BN, C), lambda i: (i, 0)),
        out_shape=jax.ShapeDtypeStruct((N, C), jnp.float32),
    )(p, g3, w1big, b1big, w2, b2r)


def kernel(x_in, neighbors_index, neighbors_row_splits, in_features, W1, b1, W2, b2):
    table = in_features[0]
    x0 = x_in[:, 0]
    x1 = x_in[:, 1]
    scale = jnp.float32(1.0 / DEG)
    g, p = _sc_gather(neighbors_index, x0, x1, table)
    g3 = g.reshape(DEG, N, C)
    p2 = p.reshape(N, 128)
    w1big = jnp.einsum("kK,fm->kfKm", jnp.eye(DEG, dtype=jnp.float32), W1)
    w1big = w1big.reshape(4 * DEG, DEG * H)
    b1big = jnp.tile(b1, DEG).reshape(1, DEG * H)
    out = _tc_mlp(p2, g3, w1big, b1big, W2 * scale, (b2 * scale).reshape(1, C))
    return out[None]


# speedup vs baseline: 3.1564x; 1.1232x over previous
import jax
import jax.numpy as jnp
import numpy as np
from jax import lax
from jax.experimental import pallas as pl
from jax.experimental.pallas import tpu as pltpu
from jax.experimental.pallas import tpu_sc as plsc

N = 10000
DEG = 32
E = N * DEG
C = 128
H = 32
KH = DEG // 2
W = C // 2

NC = 2
NS = 16
NW = NC * NS
L = 16

EDGES_PER_W = E // NW
CHUNK = 80
NCHUNKS = EDGES_PER_W // CHUNK
NPAIR = (NCHUNKS - 1) // 2


def _sc_gather_body(idx_hbm, x0_hbm, x1_hbm, table_hbm, g_hbm, p_hbm,
                    idx_v, gb0, gb1, db0, db1, ab0, ab1, x0_v, x1_v,
                    sem0, sem1, wsem0, wsem1):
    wid = lax.axis_index("s") * NC + lax.axis_index("c")
    base = wid * EDGES_PER_W

    pltpu.sync_copy(idx_hbm.at[pl.ds(base, EDGES_PER_W)], idx_v)
    pltpu.sync_copy(x0_hbm, x0_v)
    pltpu.sync_copy(x1_hbm, x1_v)

    lane = lax.iota(jnp.int32, L)

    def start(ci, gb, sem):
        pltpu.async_copy(table_hbm.at[idx_v.at[pl.ds(ci * CHUNK, CHUNK)]], gb, sem)

    def drain(gb, db, ab, wsem):
        pltpu.make_async_copy(gb, g_hbm.at[db], wsem).wait()
        pltpu.make_async_copy(ab, p_hbm.at[pl.ds(0, CHUNK * 4)], wsem).wait()

    def finish(ci, gb, db, ab, sem, wsem):
        cb = base + ci * CHUNK
        for g in range(CHUNK // L):
            jv = idx_v[pl.ds(ci * CHUNK + g * L, L)]
            ev = cb + g * L + lane
            nv = lax.shift_right_logical(ev, 5)
            kv = lax.bitwise_and(ev, 31)
            k2v = lax.bitwise_and(kv, 15)
            bv = lax.shift_right_logical(kv, 4)
            db[pl.ds(g * L, L)] = k2v * (2 * N) + 2 * nv + bv
            xj0 = plsc.load_gather(x0_v, [jv])
            xj1 = plsc.load_gather(x1_v, [jv])
            xi0 = plsc.load_gather(x0_v, [nv])
            xi1 = plsc.load_gather(x1_v, [nv])
            lv = (g * L + lane) * 4
            plsc.store_scatter(ab, [lv], xj0)
            plsc.store_scatter(ab, [lv + 1], xj1)
            plsc.store_scatter(ab, [lv + 2], xi0)
            plsc.store_scatter(ab, [lv + 3], xi1)
        pltpu.make_async_copy(table_hbm.at[idx_v.at[pl.ds(ci * CHUNK, CHUNK)]],
                              gb, sem).wait()
        pltpu.async_copy(gb, g_hbm.at[db], wsem)
        pltpu.async_copy(ab, p_hbm.at[pl.ds(cb * 4, CHUNK * 4)], wsem)

    start(0, gb0, sem0)
    start(1, gb1, sem1)
    finish(0, gb0, db0, ab0, sem0, wsem0)
    finish(1, gb1, db1, ab1, sem1, wsem1)

    def pair(oi, carry):
        c0 = 2 * oi + 2
        drain(gb0, db0, ab0, wsem0)
        start(c0, gb0, sem0)
        drain(gb1, db1, ab1, wsem1)
        start(c0 + 1, gb1, sem1)
        finish(c0, gb0, db0, ab0, sem0, wsem0)
        finish(c0 + 1, gb1, db1, ab1, sem1, wsem1)
        return carry

    lax.fori_loop(0, NPAIR - 1, pair, 0)
    drain(gb0, db0, ab0, wsem0)
    start(NCHUNKS - 1, gb0, sem0)
    finish(NCHUNKS - 1, gb0, db0, ab0, sem0, wsem0)
    drain(gb0, db0, ab0, wsem0)
    drain(gb1, db1, ab1, wsem1)


@jax.jit
def _sc_gather(neighbors_index, x0, x1, table_p):
    kern = pl.kernel(
        _sc_gather_body,
        out_type=(
            jax.ShapeDtypeStruct((2 * KH * N, W), jnp.float32),
            jax.ShapeDtypeStruct((4 * E,), jnp.float32),
        ),
        mesh=plsc.VectorSubcoreMesh(core_axis_name="c", subcore_axis_name="s"),
        compiler_params=pltpu.CompilerParams(
            use_tc_tiling_on_sc=False, needs_layout_passes=False),
        scratch_types=[
            pltpu.VMEM((EDGES_PER_W,), jnp.int32),
            pltpu.VMEM((CHUNK, W), jnp.float32),
            pltpu.VMEM((CHUNK, W), jnp.float32),
            pltpu.VMEM((CHUNK,), jnp.int32),
            pltpu.VMEM((CHUNK,), jnp.int32),
            pltpu.VMEM((CHUNK * 4,), jnp.float32),
            pltpu.VMEM((CHUNK * 4,), jnp.float32),
            pltpu.VMEM((N,), jnp.float32),
            pltpu.VMEM((N,), jnp.float32),
            pltpu.SemaphoreType.DMA,
            pltpu.SemaphoreType.DMA,
            pltpu.SemaphoreType.DMA,
            pltpu.SemaphoreType.DMA,
        ],
    )
    return kern(neighbors_index, x0, x1, table_p)


BN = 400
NBLK = N // BN


def _tc_body(p_ref, g_ref, w1_ref, b1_ref, w2e_ref, w2o_ref,
             b2e_ref, b2o_ref, perm_ref, out_ref):
    hh = jnp.dot(p_ref[...], w1_ref[...], preferred_element_type=jnp.float32)
    hh = hh + b1_ref[...]
    hh = 0.5 * hh * (1.0 + lax.erf(hh * 0.7071067811865475))
    w2e = w2e_ref[...]
    w2o = w2o_ref[...]
    b2e = b2e_ref[...]
    b2o = b2o_ref[...]
    acc_e = jnp.zeros((BN, C), jnp.float32)
    acc_o = jnp.zeros((BN, C), jnp.float32)
    for k2 in range(KH):
        h_ab = hh[:, k2 * 64:(k2 + 1) * 64]
        rep_e = jnp.dot(h_ab, w2e, preferred_element_type=jnp.float32) + b2e
        rep_o = jnp.dot(h_ab, w2o, preferred_element_type=jnp.float32) + b2o
        ui = lax.bitcast_convert_type(g_ref[k2], jnp.int32)
        ge = lax.bitcast_convert_type(lax.shift_left(ui, 16), jnp.float32)
        go = lax.bitcast_convert_type(
            lax.bitwise_and(ui, jnp.int32(-65536)), jnp.float32)
        acc_e = acc_e + rep_e * ge
        acc_o = acc_o + rep_o * go
    ev = acc_e[:, :W] + acc_e[:, W:]
    od = acc_o[:, :W] + acc_o[:, W:]
    cat = jnp.concatenate([ev, od], axis=1)
    out_ref[...] = jnp.dot(cat, perm_ref[...], preferred_element_type=jnp.float32)


@jax.jit
def _tc_mlp(p2, g3, w1big, b1big, w2e2, w2o2, b2e2, b2o2, perm):
    return pl.pallas_call(
        _tc_body,
        grid=(NBLK,),
        in_specs=[
            pl.BlockSpec((BN, 128), lambda i: (i, 0)),
            pl.BlockSpec((KH, BN, C), lambda i: (0, i, 0)),
            pl.BlockSpec((128, DEG * H), lambda i: (0, 0)),
            pl.BlockSpec((1, DEG * H), lambda i: (0, 0)),
            pl.BlockSpec((2 * H, C), lambda i: (0, 0)),
            pl.BlockSpec((2 * H, C), lambda i: (0, 0)),
            pl.BlockSpec((1, C), lambda i: (0, 0)),
            pl.BlockSpec((1, C), lambda i: (0, 0)),
            pl.BlockSpec((C, C), lambda i: (0, 0)),
        ],
        out_specs=pl.BlockSpec((BN, C), lambda i: (i, 0)),
        out_shape=jax.ShapeDtypeStruct((N, C), jnp.float32),
    )(p2, g3, w1big, b1big, w2e2, w2o2, b2e2, b2o2, perm)


def _w1_col_perm():
    k2 = np.arange(KH)[:, None, None]
    b = np.arange(2)[None, :, None]
    m = np.arange(H)[None, None, :]
    return ((b * KH + k2) * H + m).reshape(-1)


_PERM_NP = np.zeros((C, C), np.float32)
_PERM_NP[np.arange(W), 2 * np.arange(W)] = 1.0
_PERM_NP[W + np.arange(W), 2 * np.arange(W) + 1] = 1.0


def kernel(x_in, neighbors_index, neighbors_row_splits, in_features, W1, b1, W2, b2):
    table_p = lax.bitcast_convert_type(
        in_features[0].astype(jnp.bfloat16).reshape(N, W, 2), jnp.float32)
    x0 = x_in[:, 0]
    x1 = x_in[:, 1]
    scale = jnp.float32(1.0 / DEG)
    g, p = _sc_gather(neighbors_index, x0, x1, table_p)
    g3 = g.reshape(KH, N, C)
    p2 = p.reshape(N, 128)
    w1big = jnp.einsum("kK,fm->kfKm", jnp.eye(DEG, dtype=jnp.float32), W1)
    w1big = w1big.reshape(4 * DEG, DEG * H)[:, _w1_col_perm()]
    b1big = jnp.tile(b1, DEG).reshape(1, DEG * H)
    w2s = W2 * scale
    b2s = b2 * scale
    eye2 = jnp.eye(2, dtype=jnp.float32)
    w2e2 = jnp.kron(eye2, w2s[:, 0::2])
    w2o2 = jnp.kron(eye2, w2s[:, 1::2])
    b2e2 = jnp.tile(b2s[0::2], 2).reshape(1, C)
    b2o2 = jnp.tile(b2s[1::2], 2).reshape(1, C)
    perm = jnp.asarray(_PERM_NP)
    out = _tc_mlp(p2, g3, w1big, b1big, w2e2, w2o2, b2e2, b2o2, perm)
    return out[None]
